# baseline (speedup 1.0000x reference)
import functools

import jax
import jax.numpy as jnp
from jax import lax
from jax.experimental import pallas as pl
from jax.experimental.pallas import tpu as pltpu
from jax.experimental.pallas import tpu_sc as plsc

_B = 16384
_D = 32
_NC = 2
_NS = 16
_NW = _NC * _NS
_BPW = _B // _NW
_CH = 128
_NCH = _BPW // _CH
_GPC = _CH // 16


def _sc_body(uidx_hbm, iidx_hbm, emb_u_hbm, emb_i_hbm, out_hbm,
             uidx_v, iidx_v, urows, irows, dot_v, sem):
    c = lax.axis_index("c")
    s = lax.axis_index("s")
    wid = s * _NC + c
    row0 = wid * _NCH
    pltpu.sync_copy(uidx_hbm.at[pl.ds(row0, _NCH)], uidx_v)
    pltpu.sync_copy(iidx_hbm.at[pl.ds(row0, _NCH)], iidx_v)

    copies = []
    for j in range(_NCH):
        copies.append(pltpu.async_copy(emb_u_hbm.at[uidx_v.at[j]],
                                       urows[j], sem))
        copies.append(pltpu.async_copy(emb_i_hbm.at[iidx_v.at[j]],
                                       irows[j], sem))
    for cp in copies:
        cp.wait()

    lanes = lax.iota(jnp.int32, 16)

    for j in range(_NCH):
        def group_body(g, _, j=j):
            rv = g * 16 + lanes
            acc = jnp.zeros((16,), jnp.float32)
            for d in range(_D):
                dv = jnp.full((16,), d, jnp.int32)
                uv = plsc.load_gather(urows[j], [rv, dv])
                iv = plsc.load_gather(irows[j], [rv, dv])
                acc = acc + uv * iv
            dot_v[pl.ds((j * _GPC + g) * 16, 16)] = acc
            return ()

        lax.fori_loop(0, _GPC, group_body, ())
    pltpu.sync_copy(dot_v, out_hbm.at[pl.ds(wid * _BPW, _BPW)])


@functools.partial(
    pl.kernel,
    out_type=jax.ShapeDtypeStruct((_B,), jnp.float32),
    mesh=plsc.VectorSubcoreMesh(core_axis_name="c", subcore_axis_name="s",
                                num_cores=_NC, num_subcores=_NS),
    compiler_params=pltpu.CompilerParams(needs_layout_passes=False,
                                         use_tc_tiling_on_sc=False),
    scratch_types=[
        pltpu.VMEM((_NCH, _CH), jnp.int32),
        pltpu.VMEM((_NCH, _CH), jnp.int32),
        [pltpu.VMEM((_CH, _D), jnp.float32) for _ in range(_NCH)],
        [pltpu.VMEM((_CH, _D), jnp.float32) for _ in range(_NCH)],
        pltpu.VMEM((_BPW,), jnp.float32),
        pltpu.SemaphoreType.DMA,
    ],
)
def _sc_dot(uidx_hbm, iidx_hbm, emb_u_hbm, emb_i_hbm, out_hbm,
            uidx_v, iidx_v, urows, irows, dot_v, sem):
    _sc_body(uidx_hbm, iidx_hbm, emb_u_hbm, emb_i_hbm, out_hbm,
             uidx_v, iidx_v, urows, irows, dot_v, sem)


def _tc_body(dot_ref, labels_ref, rating_ref, loss_ref):
    x = dot_ref[...]
    r = jax.nn.sigmoid(x)
    rating_ref[...] = r
    y = labels_ref[...]
    t = jnp.maximum(r, 0.0) - r * y + jnp.log1p(jnp.exp(-jnp.abs(r)))
    loss_ref[0, 0] = jnp.sum(t) / _B


def kernel(user_indices, item_indices, labels, emb_user, emb_item):
    uidx2d = user_indices.reshape(_B // _CH, _CH)
    iidx2d = item_indices.reshape(_B // _CH, _CH)
    dot = _sc_dot(uidx2d, iidx2d, emb_user, emb_item)

    rating2d, loss11 = pl.pallas_call(
        _tc_body,
        out_shape=[
            jax.ShapeDtypeStruct((_B // 128, 128), jnp.float32),
            jax.ShapeDtypeStruct((1, 1), jnp.float32),
        ],
        out_specs=[
            pl.BlockSpec(memory_space=pltpu.VMEM),
            pl.BlockSpec(memory_space=pltpu.SMEM),
        ],
    )(dot.reshape(_B // 128, 128), labels.reshape(_B // 128, 128))

    rating = rating2d.reshape(_B)
    loss = loss11.reshape(())
    return (loss, loss, rating, labels)


# speedup vs baseline: 2.1701x; 2.1701x over previous
import functools

import jax
import jax.numpy as jnp
from jax import lax
from jax.experimental import pallas as pl
from jax.experimental.pallas import tpu as pltpu
from jax.experimental.pallas import tpu_sc as plsc

_B = 16384
_D = 32
_NC = 2
_NS = 16
_NW = _NC * _NS
_BPW = _B // _NW
_CH = 16
_NCHK = _BPW // _CH


def _sc_body(uidx_hbm, iidx_hbm, emb_u3_hbm, emb_i3_hbm, out_hbm,
             uidx_v, iidx_v, slab_u, slab_i, dot_v, sem_u, sem_i):
    c = lax.axis_index("c")
    s = lax.axis_index("s")
    wid = s * _NC + c
    base = wid * _BPW
    pltpu.sync_copy(uidx_hbm.at[pl.ds(base, _BPW)], uidx_v)
    pltpu.sync_copy(iidx_hbm.at[pl.ds(base, _BPW)], iidx_v)

    lanes = lax.iota(jnp.int32, 16)

    def chunk(k, _):
        tu = lax.div(uidx_v[pl.ds(k * _CH, 16)], 8)
        ti = lax.div(iidx_v[pl.ds(k * _CH, 16)], 8)
        for i in range(_CH):
            pltpu.async_copy(emb_u3_hbm.at[tu[i]], slab_u.at[i], sem_u)
            pltpu.async_copy(emb_i3_hbm.at[ti[i]], slab_i.at[i], sem_i)
        pltpu.make_async_copy(emb_u3_hbm.at[pl.ds(0, _CH)], slab_u,
                              sem_u).wait()
        pltpu.make_async_copy(emb_i3_hbm.at[pl.ds(0, _CH)], slab_i,
                              sem_i).wait()
        su = lax.rem(plsc.load_gather(uidx_v, [k * _CH + lanes]), 8)
        si = lax.rem(plsc.load_gather(iidx_v, [k * _CH + lanes]), 8)
        acc = jnp.zeros((16,), jnp.float32)
        for d in range(_D):
            dv = jnp.full((16,), d, jnp.int32)
            uv = plsc.load_gather(slab_u, [lanes, su, dv])
            iv = plsc.load_gather(slab_i, [lanes, si, dv])
            acc = acc + uv * iv
        dot_v[pl.ds(k * _CH, 16)] = acc
        return ()

    lax.fori_loop(0, _NCHK, chunk, ())
    pltpu.sync_copy(dot_v, out_hbm.at[pl.ds(base, _BPW)])


@functools.partial(
    pl.kernel,
    out_type=jax.ShapeDtypeStruct((_B,), jnp.float32),
    mesh=plsc.VectorSubcoreMesh(core_axis_name="c", subcore_axis_name="s",
                                num_cores=_NC, num_subcores=_NS),
    compiler_params=pltpu.CompilerParams(needs_layout_passes=False),
    scratch_types=[
        pltpu.VMEM((_BPW,), jnp.int32),
        pltpu.VMEM((_BPW,), jnp.int32),
        pltpu.VMEM((_CH, 8, _D), jnp.float32),
        pltpu.VMEM((_CH, 8, _D), jnp.float32),
        pltpu.VMEM((_BPW,), jnp.float32),
        pltpu.SemaphoreType.DMA,
        pltpu.SemaphoreType.DMA,
    ],
)
def _sc_dot(uidx_hbm, iidx_hbm, emb_u3_hbm, emb_i3_hbm, out_hbm,
            uidx_v, iidx_v, slab_u, slab_i, dot_v, sem_u, sem_i):
    _sc_body(uidx_hbm, iidx_hbm, emb_u3_hbm, emb_i3_hbm, out_hbm,
             uidx_v, iidx_v, slab_u, slab_i, dot_v, sem_u, sem_i)


def _tc_body(dot_ref, labels_ref, rating_ref, loss_ref):
    x = dot_ref[...]
    r = jax.nn.sigmoid(x)
    rating_ref[...] = r
    y = labels_ref[...]
    t = jnp.maximum(r, 0.0) - r * y + jnp.log1p(jnp.exp(-jnp.abs(r)))
    loss_ref[0, 0] = jnp.sum(t) / _B


def kernel(user_indices, item_indices, labels, emb_user, emb_item):
    emb_u3 = emb_user.reshape(125000, 8, _D)
    emb_i3 = emb_item.reshape(125000, 8, _D)
    dot = _sc_dot(user_indices, item_indices, emb_u3, emb_i3)

    rating2d, loss11 = pl.pallas_call(
        _tc_body,
        out_shape=[
            jax.ShapeDtypeStruct((_B // 128, 128), jnp.float32),
            jax.ShapeDtypeStruct((1, 1), jnp.float32),
        ],
        out_specs=[
            pl.BlockSpec(memory_space=pltpu.VMEM),
            pl.BlockSpec(memory_space=pltpu.SMEM),
        ],
    )(dot.reshape(_B // 128, 128), labels.reshape(_B // 128, 128))

    rating = rating2d.reshape(_B)
    loss = loss11.reshape(())
    return (loss, loss, rating, labels)
